# baseline (speedup 1.0000x reference)
import functools

import jax
import jax.numpy as jnp
from jax import lax
from jax.experimental import pallas as pl
from jax.experimental.pallas import tpu as pltpu
from jax.experimental.pallas import tpu_sc as plsc

_C = 96
_H = 224
_W = 224
_BINS = 64
_N = 4
_ROWS = _N * _C
_HW = _H * _W
_SUB = 392
_LANES = 128
_TOT = _ROWS * _HW


def _minmax_body(x_ref, mn_ref, mx_ref):
    i = pl.program_id(0)
    bm = jnp.min(x_ref[...])
    bM = jnp.max(x_ref[...])

    @pl.when(i == 0)
    def _init():
        mn_ref[0, 0] = bm
        mx_ref[0, 0] = bM

    @pl.when(i > 0)
    def _acc():
        mn_ref[0, 0] = jnp.minimum(mn_ref[0, 0], bm)
        mx_ref[0, 0] = jnp.maximum(mx_ref[0, 0], bM)


def _minmax(x_r, interpret=False):
    grid = 48
    blk = _ROWS // grid
    return pl.pallas_call(
        _minmax_body,
        grid=(grid,),
        in_specs=[pl.BlockSpec((blk, _SUB, _LANES), lambda i: (i, 0, 0))],
        out_specs=[
            pl.BlockSpec(memory_space=pltpu.SMEM),
            pl.BlockSpec(memory_space=pltpu.SMEM),
        ],
        out_shape=[
            jax.ShapeDtypeStruct((1, 1), jnp.float32),
            jax.ShapeDtypeStruct((1, 1), jnp.float32),
        ],
        compiler_params=pltpu.CompilerParams(
            dimension_semantics=("arbitrary",)),
        interpret=interpret,
    )(x_r)


def _scan_body(x_ref, tau_ref, p_ref, c_ref, carry_ref, cnt_ref):
    i = pl.program_id(0)

    @pl.when(i == 0)
    def _init():
        carry_ref[0] = 0.0
        cnt_ref[...] = jnp.zeros((1, _BINS), jnp.int32)

    blk = x_ref[0]

    t = jnp.tanh(blk)
    iu = lax.broadcasted_iota(jnp.int32, (_LANES, _LANES), 0)
    ju = lax.broadcasted_iota(jnp.int32, (_LANES, _LANES), 1)
    U = jnp.where(iu <= ju, 1.0, 0.0)
    cs = jnp.dot(t, U, preferred_element_type=jnp.float32,
                 precision=lax.Precision.HIGHEST)
    rt = cs[:, _LANES - 1:_LANES]
    il = lax.broadcasted_iota(jnp.int32, (_SUB, _SUB), 0)
    jl = lax.broadcasted_iota(jnp.int32, (_SUB, _SUB), 1)
    Ls = jnp.where(jl < il, 1.0, 0.0)
    ro = jnp.dot(Ls, rt, preferred_element_type=jnp.float32,
                 precision=lax.Precision.HIGHEST)
    carry = carry_ref[0]
    p_ref[0] = cs + ro + carry
    carry_ref[0] = carry + jnp.sum(t)

    ge = [jnp.float32(_HW)]
    for b in range(1, _BINS):
        ge.append(jnp.sum(jnp.where(blk >= tau_ref[b], 1.0, 0.0)))
    g = jnp.stack(ge)
    gnext = jnp.concatenate([g[1:], jnp.zeros((1,), jnp.float32)])
    hist = (g - gnext).astype(jnp.int32)
    new_cnt = cnt_ref[...] + hist.reshape(1, _BINS)
    cnt_ref[...] = new_cnt
    c_ref[0] = new_cnt


def _scan_counts(x_r, tau, interpret=False):
    return pl.pallas_call(
        _scan_body,
        grid=(_ROWS,),
        in_specs=[
            pl.BlockSpec((1, _SUB, _LANES), lambda i: (i, 0, 0)),
            pl.BlockSpec(memory_space=pltpu.SMEM),
        ],
        out_specs=[
            pl.BlockSpec((1, _SUB, _LANES), lambda i: (i, 0, 0)),
            pl.BlockSpec((1, 1, _BINS), lambda i: (i, 0, 0)),
        ],
        out_shape=[
            jax.ShapeDtypeStruct((_ROWS, _SUB, _LANES), jnp.float32),
            jax.ShapeDtypeStruct((_ROWS, 1, _BINS), jnp.int32),
        ],
        scratch_shapes=[
            pltpu.SMEM((1,), jnp.float32),
            pltpu.VMEM((1, _BINS), jnp.int32),
        ],
        compiler_params=pltpu.CompilerParams(
            dimension_semantics=("arbitrary",)),
        interpret=interpret,
    )(x_r, tau)


_NW = 24


def _combine_sc(p_flat, idx, msk, cf):
    mesh = plsc.VectorSubcoreMesh(core_axis_name="c", subcore_axis_name="s")

    @functools.partial(
        pl.kernel,
        out_type=jax.ShapeDtypeStruct((_ROWS,), jnp.float32),
        mesh=mesh,
        scratch_types=[
            pltpu.VMEM((16, 128), jnp.int32),
            pltpu.VMEM((16, 128), jnp.float32),
            pltpu.VMEM((8, 128), jnp.float32),
            pltpu.VMEM((16, 128), jnp.float32),
            pltpu.VMEM((16,), jnp.float32),
            pltpu.SemaphoreType.DMA,
        ],
    )
    def kc(p_hbm, idx_hbm, msk_hbm, cf_hbm, out_hbm,
           idx_v, msk_v, cf_v, vals_v, acc_v, sem):
        w = lax.axis_index("s") * 2 + lax.axis_index("c")

        @pl.when(w < _NW)
        def _():
            pltpu.sync_copy(idx_hbm.at[w], idx_v)
            pltpu.sync_copy(msk_hbm.at[w], msk_v)
            pltpu.sync_copy(cf_hbm.at[w], cf_v)
            cps = [pltpu.async_copy(p_hbm.at[idx_v.at[ch]], vals_v.at[ch], sem)
                   for ch in range(16)]
            for cp in cps:
                cp.wait()

            def body(b, acc):
                r = b // 8
                off = (b % 8) * 16
                va = vals_v[r, pl.ds(off, 16)]
                vb = vals_v[r + 8, pl.ds(off, 16)]
                ma = msk_v[r, pl.ds(off, 16)]
                mb = msk_v[r + 8, pl.ds(off, 16)]
                cfv = cf_v[r, pl.ds(off, 16)]
                return acc + cfv * (vb * mb - va * ma)

            acc_v[...] = lax.fori_loop(0, _BINS, body,
                                       jnp.zeros((16,), jnp.float32))
            pltpu.sync_copy(acc_v, out_hbm.at[pl.ds(16 * w, 16)])

    return kc(p_flat, idx, msk, cf)


def _run(x, coeff, interpret=False):
    x_r = x.reshape(_ROWS, _SUB, _LANES)
    mn, mx = _minmax(x_r, interpret)
    tau = jnp.linspace(mn[0, 0], mx[0, 0], _BINS + 1)
    P, Ccum = _scan_counts(x_r, tau, interpret)
    Q = jnp.concatenate(
        [jnp.zeros((1, _BINS), jnp.int32), Ccum.reshape(_ROWS, _BINS)], axis=0)
    QT = Q.T
    QTw = jnp.stack([lax.slice(QT, (0, 16 * w), (_BINS, 16 * w + 17))
                     for w in range(_NW)])
    qa = QTw[:, :, 0:16].reshape(_NW, 1024)
    qb = QTw[:, :, 1:17].reshape(_NW, 1024)
    qcat = jnp.concatenate([qa, qb], axis=1)
    idx = jnp.maximum(qcat - 1, 0).reshape(_NW, 16, 128)
    msk = (qcat > 0).astype(jnp.float32).reshape(_NW, 16, 128)
    coeffT = coeff.T
    cf = jnp.stack([lax.slice(coeffT, (0, (16 * w) % _C),
                              (_BINS, (16 * w) % _C + 16))
                    for w in range(_NW)]).reshape(_NW, 8, 128)
    if interpret:
        Pf = P.reshape(-1)
        g = jnp.where(Q > 0, jnp.take(Pf, jnp.maximum(Q - 1, 0)), 0.0)
        T = g[1:] - g[:-1]
        return jnp.sum(T.reshape(_N, _C, _BINS) * coeff[None], axis=2)
    z = _combine_sc(P.reshape(-1), idx, msk, cf)
    return z.reshape(_N, _C)


def kernel(x, coeff):
    return _run(x, coeff, interpret=False)


# speedup vs baseline: 1.2087x; 1.2087x over previous
import functools

import jax
import jax.numpy as jnp
from jax import lax
from jax.experimental import pallas as pl
from jax.experimental.pallas import tpu as pltpu
from jax.experimental.pallas import tpu_sc as plsc

_C = 96
_H = 224
_W = 224
_BINS = 64
_N = 4
_ROWS = _N * _C
_HW = _H * _W
_SUB = 392
_LANES = 128
_TOT = _ROWS * _HW


def _minmax_body(x_ref, mn_ref, mx_ref):
    i = pl.program_id(0)
    bm = jnp.min(x_ref[...])
    bM = jnp.max(x_ref[...])

    @pl.when(i == 0)
    def _init():
        mn_ref[0, 0] = bm
        mx_ref[0, 0] = bM

    @pl.when(i > 0)
    def _acc():
        mn_ref[0, 0] = jnp.minimum(mn_ref[0, 0], bm)
        mx_ref[0, 0] = jnp.maximum(mx_ref[0, 0], bM)


def _minmax(x_r, interpret=False):
    grid = 48
    blk = _ROWS // grid
    return pl.pallas_call(
        _minmax_body,
        grid=(grid,),
        in_specs=[pl.BlockSpec((blk, _SUB, _LANES), lambda i: (i, 0, 0))],
        out_specs=[
            pl.BlockSpec(memory_space=pltpu.SMEM),
            pl.BlockSpec(memory_space=pltpu.SMEM),
        ],
        out_shape=[
            jax.ShapeDtypeStruct((1, 1), jnp.float32),
            jax.ShapeDtypeStruct((1, 1), jnp.float32),
        ],
        compiler_params=pltpu.CompilerParams(
            dimension_semantics=("arbitrary",)),
        interpret=interpret,
    )(x_r)


def _scan_body(x_ref, tau_ref, p_ref, c_ref, carry_ref, cnt_ref):
    i = pl.program_id(0)

    @pl.when(i == 0)
    def _init():
        carry_ref[0] = 0.0
        cnt_ref[...] = jnp.zeros((1, _BINS), jnp.int32)

    blk = x_ref[0]

    t = jnp.tanh(blk)
    iu = lax.broadcasted_iota(jnp.int32, (_LANES, _LANES), 0)
    ju = lax.broadcasted_iota(jnp.int32, (_LANES, _LANES), 1)
    U = jnp.where(iu <= ju, 1.0, 0.0)
    cs = jnp.dot(t, U, preferred_element_type=jnp.float32,
                 precision=lax.Precision.HIGHEST)
    rt = cs[:, _LANES - 1:_LANES]
    rtt = jnp.transpose(rt)
    rtp = jnp.concatenate(
        [rtt, jnp.zeros((1, 512 - _SUB), jnp.float32)], axis=1)
    r4 = jnp.concatenate(
        [rtp[:, 128 * g:128 * (g + 1)] for g in range(4)], axis=0)
    s = r4
    k = 1
    while k < 128:
        s = s + jnp.concatenate(
            [jnp.zeros((4, k), jnp.float32), s[:, :128 - k]], axis=1)
        k *= 2
    gt = s[:, 127:128]
    go = jnp.concatenate(
        [jnp.zeros((1, 1), jnp.float32),
         gt[0:1], gt[0:1] + gt[1:2], gt[0:1] + gt[1:2] + gt[2:3]], axis=0)
    excl4 = s + go - r4
    exr = jnp.concatenate([excl4[g:g + 1] for g in range(4)], axis=1)
    ro = jnp.transpose(exr[:, :_SUB])
    carry = carry_ref[0]
    p_ref[0] = cs + ro + carry
    carry_ref[0] = p_ref[0, _SUB - 1, _LANES - 1]

    ge = [jnp.float32(_HW)]
    for b in range(1, _BINS):
        ge.append(jnp.sum(jnp.where(blk >= tau_ref[b], 1.0, 0.0)))
    g = jnp.stack(ge)
    gnext = jnp.concatenate([g[1:], jnp.zeros((1,), jnp.float32)])
    hist = (g - gnext).astype(jnp.int32)
    new_cnt = cnt_ref[...] + hist.reshape(1, _BINS)
    cnt_ref[...] = new_cnt
    c_ref[0] = new_cnt


def _scan_counts(x_r, tau, interpret=False):
    return pl.pallas_call(
        _scan_body,
        grid=(_ROWS,),
        in_specs=[
            pl.BlockSpec((1, _SUB, _LANES), lambda i: (i, 0, 0)),
            pl.BlockSpec(memory_space=pltpu.SMEM),
        ],
        out_specs=[
            pl.BlockSpec((1, _SUB, _LANES), lambda i: (i, 0, 0)),
            pl.BlockSpec((1, 1, _BINS), lambda i: (i, 0, 0)),
        ],
        out_shape=[
            jax.ShapeDtypeStruct((_ROWS, _SUB, _LANES), jnp.float32),
            jax.ShapeDtypeStruct((_ROWS, 1, _BINS), jnp.int32),
        ],
        scratch_shapes=[
            pltpu.SMEM((1,), jnp.float32),
            pltpu.VMEM((1, _BINS), jnp.int32),
        ],
        compiler_params=pltpu.CompilerParams(
            dimension_semantics=("arbitrary",)),
        interpret=interpret,
    )(x_r, tau)


_NW = 24


def _combine_sc(p_flat, idx, msk, cf):
    mesh = plsc.VectorSubcoreMesh(core_axis_name="c", subcore_axis_name="s")

    @functools.partial(
        pl.kernel,
        out_type=jax.ShapeDtypeStruct((_ROWS,), jnp.float32),
        mesh=mesh,
        scratch_types=[
            pltpu.VMEM((16, 128), jnp.int32),
            pltpu.VMEM((16, 128), jnp.float32),
            pltpu.VMEM((8, 128), jnp.float32),
            pltpu.VMEM((16, 128), jnp.float32),
            pltpu.VMEM((16,), jnp.float32),
            pltpu.SemaphoreType.DMA,
        ],
    )
    def kc(p_hbm, idx_hbm, msk_hbm, cf_hbm, out_hbm,
           idx_v, msk_v, cf_v, vals_v, acc_v, sem):
        w = lax.axis_index("s") * 2 + lax.axis_index("c")

        @pl.when(w < _NW)
        def _():
            pltpu.sync_copy(idx_hbm.at[w], idx_v)
            pltpu.sync_copy(msk_hbm.at[w], msk_v)
            pltpu.sync_copy(cf_hbm.at[w], cf_v)
            cps = [pltpu.async_copy(p_hbm.at[idx_v.at[ch]], vals_v.at[ch], sem)
                   for ch in range(16)]
            for cp in cps:
                cp.wait()

            def body(b, acc):
                r = b // 8
                off = (b % 8) * 16
                va = vals_v[r, pl.ds(off, 16)]
                vb = vals_v[r + 8, pl.ds(off, 16)]
                ma = msk_v[r, pl.ds(off, 16)]
                mb = msk_v[r + 8, pl.ds(off, 16)]
                cfv = cf_v[r, pl.ds(off, 16)]
                return acc + cfv * (vb * mb - va * ma)

            acc_v[...] = lax.fori_loop(0, _BINS, body,
                                       jnp.zeros((16,), jnp.float32))
            pltpu.sync_copy(acc_v, out_hbm.at[pl.ds(16 * w, 16)])

    return kc(p_flat, idx, msk, cf)


def _run(x, coeff, interpret=False):
    x_r = x.reshape(_ROWS, _SUB, _LANES)
    mn, mx = _minmax(x_r, interpret)
    tau = jnp.linspace(mn[0, 0], mx[0, 0], _BINS + 1)
    P, Ccum = _scan_counts(x_r, tau, interpret)
    Q = jnp.concatenate(
        [jnp.zeros((1, _BINS), jnp.int32), Ccum.reshape(_ROWS, _BINS)], axis=0)
    QT = Q.T
    QTw = jnp.stack([lax.slice(QT, (0, 16 * w), (_BINS, 16 * w + 17))
                     for w in range(_NW)])
    qa = QTw[:, :, 0:16].reshape(_NW, 1024)
    qb = QTw[:, :, 1:17].reshape(_NW, 1024)
    qcat = jnp.concatenate([qa, qb], axis=1)
    idx = jnp.maximum(qcat - 1, 0).reshape(_NW, 16, 128)
    msk = (qcat > 0).astype(jnp.float32).reshape(_NW, 16, 128)
    coeffT = coeff.T
    cf = jnp.stack([lax.slice(coeffT, (0, (16 * w) % _C),
                              (_BINS, (16 * w) % _C + 16))
                    for w in range(_NW)]).reshape(_NW, 8, 128)
    if interpret:
        Pf = P.reshape(-1)
        g = jnp.where(Q > 0, jnp.take(Pf, jnp.maximum(Q - 1, 0)), 0.0)
        T = g[1:] - g[:-1]
        return jnp.sum(T.reshape(_N, _C, _BINS) * coeff[None], axis=2)
    z = _combine_sc(P.reshape(-1), idx, msk, cf)
    return z.reshape(_N, _C)


def kernel(x, coeff):
    return _run(x, coeff, interpret=False)


# speedup vs baseline: 1.4770x; 1.2219x over previous
import functools

import jax
import jax.numpy as jnp
from jax import lax
from jax.experimental import pallas as pl
from jax.experimental.pallas import tpu as pltpu
from jax.experimental.pallas import tpu_sc as plsc

_C = 96
_H = 224
_W = 224
_BINS = 64
_N = 4
_ROWS = _N * _C
_HW = _H * _W
_SUB = 392
_LANES = 128
_TOT = _ROWS * _HW


def _minmax_body(x_ref, mn_ref, mx_ref):
    i = pl.program_id(0)
    bm = jnp.min(x_ref[...])
    bM = jnp.max(x_ref[...])

    @pl.when(i == 0)
    def _init():
        mn_ref[0, 0] = bm
        mx_ref[0, 0] = bM

    @pl.when(i > 0)
    def _acc():
        mn_ref[0, 0] = jnp.minimum(mn_ref[0, 0], bm)
        mx_ref[0, 0] = jnp.maximum(mx_ref[0, 0], bM)


def _minmax(x_r, interpret=False):
    grid = 48
    blk = _ROWS // grid
    return pl.pallas_call(
        _minmax_body,
        grid=(grid,),
        in_specs=[pl.BlockSpec((blk, _H, _W), lambda i: (i, 0, 0))],
        out_specs=[
            pl.BlockSpec(memory_space=pltpu.SMEM),
            pl.BlockSpec(memory_space=pltpu.SMEM),
        ],
        out_shape=[
            jax.ShapeDtypeStruct((1, 1), jnp.float32),
            jax.ShapeDtypeStruct((1, 1), jnp.float32),
        ],
        compiler_params=pltpu.CompilerParams(
            dimension_semantics=("arbitrary",)),
        interpret=interpret,
    )(x_r)


_RBLK = 8


def _row_scan(t):
    iu = lax.broadcasted_iota(jnp.int32, (_W, _W), 0)
    ju = lax.broadcasted_iota(jnp.int32, (_W, _W), 1)
    U = jnp.where(iu <= ju, 1.0, 0.0)
    cs = jnp.dot(t, U, preferred_element_type=jnp.float32,
                 precision=lax.Precision.HIGHEST)
    rt = cs[:, _W - 1:_W]
    rtt = jnp.transpose(rt)
    rtp = jnp.concatenate(
        [rtt, jnp.zeros((1, 256 - _H), jnp.float32)], axis=1)
    r2 = jnp.concatenate(
        [rtp[:, 128 * g:128 * (g + 1)] for g in range(2)], axis=0)
    s = r2
    k = 1
    while k < 128:
        s = s + jnp.concatenate(
            [jnp.zeros((2, k), jnp.float32), s[:, :128 - k]], axis=1)
        k *= 2
    gt = s[:, 127:128]
    go = jnp.concatenate(
        [jnp.zeros((1, 1), jnp.float32), gt[0:1]], axis=0)
    excl2 = s + go - r2
    exr = jnp.concatenate([excl2[g:g + 1] for g in range(2)], axis=1)
    ro = jnp.transpose(exr[:, :_H])
    return cs + ro


_WP = 256


def _row_counts(blk, tau_ref):
    ge = [jnp.float32(_HW)]
    for b in range(1, _BINS):
        ge.append(jnp.sum(jnp.where(blk >= tau_ref[b], 1.0, 0.0)))
    g = jnp.stack(ge)
    gnext = jnp.concatenate([g[1:], jnp.zeros((1,), jnp.float32)])
    return (g - gnext).astype(jnp.int32).reshape(1, _BINS)


def _scan_body(x_ref, tau_ref, p_ref, c_ref, carry_ref, cnt_ref):
    i = pl.program_id(0)

    @pl.when(i == 0)
    def _init():
        carry_ref[0] = 0.0
        cnt_ref[...] = jnp.zeros((1, _BINS), jnp.int32)
        p_ref[0, 0:1, _WP - 128:_WP] = jnp.zeros((1, 128), jnp.float32)

    for r in range(_RBLK):
        blk = x_ref[r]
        p_ref[r, :, 0:_W] = _row_scan(jnp.tanh(blk)) + carry_ref[0]
        carry_ref[0] = p_ref[r, _H - 1, _W - 1]
        new_cnt = cnt_ref[...] + _row_counts(blk, tau_ref)
        cnt_ref[...] = new_cnt
        c_ref[r] = new_cnt


def _scan_counts(x_r, tau, interpret=False):
    return pl.pallas_call(
        _scan_body,
        grid=(_ROWS // _RBLK,),
        in_specs=[
            pl.BlockSpec((_RBLK, _H, _W), lambda i: (i, 0, 0)),
            pl.BlockSpec(memory_space=pltpu.SMEM),
        ],
        out_specs=[
            pl.BlockSpec((_RBLK, _H, _WP), lambda i: (i, 0, 0)),
            pl.BlockSpec((_RBLK, 1, _BINS), lambda i: (i, 0, 0)),
        ],
        out_shape=[
            jax.ShapeDtypeStruct((_ROWS, _H, _WP), jnp.float32),
            jax.ShapeDtypeStruct((_ROWS, 1, _BINS), jnp.int32),
        ],
        scratch_shapes=[
            pltpu.SMEM((1,), jnp.float32),
            pltpu.VMEM((1, _BINS), jnp.int32),
        ],
        compiler_params=pltpu.CompilerParams(
            dimension_semantics=("arbitrary",)),
        interpret=interpret,
    )(x_r, tau)


_NW = 24


def _combine_sc(p_flat, idxa, cf):
    mesh = plsc.VectorSubcoreMesh(core_axis_name="c", subcore_axis_name="s")

    def _lane_gather(v, pidx):
        dn = lax.GatherDimensionNumbers(offset_dims=(), collapsed_slice_dims=(0,),
                                        start_index_map=(0,))
        return lax.gather(v, pidx.reshape(16, 1), dn, slice_sizes=(1,),
                          mode=lax.GatherScatterMode.PROMISE_IN_BOUNDS)

    @functools.partial(
        pl.kernel,
        out_type=jax.ShapeDtypeStruct((_ROWS,), jnp.float32),
        mesh=mesh,
        scratch_types=[
            pltpu.VMEM((9, 128), jnp.int32),
            pltpu.VMEM((8, 128), jnp.float32),
            pltpu.VMEM((9, 128), jnp.float32),
            pltpu.VMEM((16,), jnp.float32),
            pltpu.SemaphoreType.DMA,
        ],
    )
    def kc(p_hbm, idxa_hbm, cf_hbm, out_hbm,
           idxa_v, cf_v, vals_v, acc_v, sem):
        w = lax.axis_index("s") * 2 + lax.axis_index("c")

        @pl.when(w < _NW)
        def _():
            pltpu.sync_copy(idxa_hbm.at[w], idxa_v)
            pltpu.sync_copy(cf_hbm.at[w], cf_v)
            cps = [pltpu.async_copy(p_hbm.at[idxa_v.at[ch]], vals_v.at[ch],
                                    sem)
                   for ch in range(9)]
            for cp in cps:
                cp.wait()

            lane = lax.broadcasted_iota(jnp.int32, (16,), 0)
            perm = jnp.minimum(lane + 1, 15)
            last = lane == 15

            def body(b, acc):
                r = b // 8
                off = (b % 8) * 16
                va = vals_v[r, pl.ds(off, 16)]
                ex = vals_v[8, pl.ds((b // 16) * 16, 16)]
                sp = jnp.zeros((16,), jnp.int32) + b % 16
                ve = _lane_gather(ex, sp)
                vb = jnp.where(last, ve, _lane_gather(va, perm))
                cfv = cf_v[r, pl.ds(off, 16)]
                return acc + cfv * (vb - va)

            acc_v[...] = lax.fori_loop(0, _BINS, body,
                                       jnp.zeros((16,), jnp.float32))
            pltpu.sync_copy(acc_v, out_hbm.at[pl.ds(16 * w, 16)])

    return kc(p_flat, idxa, cf)


def _run(x, coeff, interpret=False):
    x_r = x.reshape(_ROWS, _H, _W)
    mn, mx = _minmax(x_r, interpret)
    tau = jnp.linspace(mn[0, 0], mx[0, 0], _BINS + 1)
    P, Ccum = _scan_counts(x_r, tau, interpret)
    Q = jnp.concatenate(
        [jnp.zeros((1, _BINS), jnp.int32), Ccum.reshape(_ROWS, _BINS)], axis=0)
    QT = Q.T
    QTw = jnp.stack([lax.slice(QT, (0, 16 * w), (_BINS, 16 * w + 17))
                     for w in range(_NW)])
    qa = QTw[:, :, 0:16].reshape(_NW, 1024)
    qe = QTw[:, :, 16]

    def _phys(q):
        qm1 = jnp.maximum(q - 1, 0)
        return jnp.where(q > 0, (qm1 // _W) * _WP + qm1 % _W, _WP - 1)

    iex = jnp.concatenate(
        [_phys(qe), jnp.full((_NW, 64), _WP - 1, jnp.int32)], axis=1)
    idxa = jnp.concatenate(
        [_phys(qa).reshape(_NW, 8, 128), iex.reshape(_NW, 1, 128)], axis=1)
    coeffT = coeff.T
    cf = jnp.stack([lax.slice(coeffT, (0, (16 * w) % _C),
                              (_BINS, (16 * w) % _C + 16))
                    for w in range(_NW)]).reshape(_NW, 8, 128)
    if interpret:
        Pf = P[:, :, 0:_W].reshape(-1)
        g = jnp.where(Q > 0, jnp.take(Pf, jnp.maximum(Q - 1, 0)), 0.0)
        T = g[1:] - g[:-1]
        return jnp.sum(T.reshape(_N, _C, _BINS) * coeff[None], axis=2)
    z = _combine_sc(P.reshape(-1), idxa, cf)
    return z.reshape(_N, _C)


def kernel(x, coeff):
    return _run(x, coeff, interpret=False)


# speedup vs baseline: 1.4921x; 1.0102x over previous
import functools

import jax
import jax.numpy as jnp
from jax import lax
from jax.experimental import pallas as pl
from jax.experimental.pallas import tpu as pltpu
from jax.experimental.pallas import tpu_sc as plsc

_C = 96
_H = 224
_W = 224
_BINS = 64
_N = 4
_ROWS = _N * _C
_HW = _H * _W
_SUB = 392
_LANES = 128
_TOT = _ROWS * _HW


def _minmax_body(x_ref, mn_ref, mx_ref):
    i = pl.program_id(0)
    bm = jnp.min(x_ref[...])
    bM = jnp.max(x_ref[...])

    @pl.when(i == 0)
    def _init():
        mn_ref[0, 0] = bm
        mx_ref[0, 0] = bM

    @pl.when(i > 0)
    def _acc():
        mn_ref[0, 0] = jnp.minimum(mn_ref[0, 0], bm)
        mx_ref[0, 0] = jnp.maximum(mx_ref[0, 0], bM)


def _minmax(x_r):
    grid = 48
    blk = _ROWS // grid
    return pl.pallas_call(
        _minmax_body,
        grid=(grid,),
        in_specs=[pl.BlockSpec((blk, _SUB, _LANES), lambda i: (i, 0, 0))],
        out_specs=[
            pl.BlockSpec(memory_space=pltpu.SMEM),
            pl.BlockSpec(memory_space=pltpu.SMEM),
        ],
        out_shape=[
            jax.ShapeDtypeStruct((1, 1), jnp.float32),
            jax.ShapeDtypeStruct((1, 1), jnp.float32),
        ],
        compiler_params=pltpu.CompilerParams(
            dimension_semantics=("arbitrary",)),
    )(x_r)


_RBLK = 8


def _row_scan(t):
    iu = lax.broadcasted_iota(jnp.int32, (_LANES, _LANES), 0)
    ju = lax.broadcasted_iota(jnp.int32, (_LANES, _LANES), 1)
    U = jnp.where(iu <= ju, 1.0, 0.0)
    cs = jnp.dot(t, U, preferred_element_type=jnp.float32,
                 precision=lax.Precision.HIGHEST)
    rt = cs[:, _LANES - 1:_LANES]
    rtt = jnp.transpose(rt)
    rtp = jnp.concatenate(
        [rtt, jnp.zeros((1, 512 - _SUB), jnp.float32)], axis=1)
    r4 = jnp.concatenate(
        [rtp[:, 128 * g:128 * (g + 1)] for g in range(4)], axis=0)
    s = r4
    k = 1
    while k < 128:
        s = s + jnp.concatenate(
            [jnp.zeros((4, k), jnp.float32), s[:, :128 - k]], axis=1)
        k *= 2
    gt = s[:, 127:128]
    go = jnp.concatenate(
        [jnp.zeros((1, 1), jnp.float32),
         gt[0:1], gt[0:1] + gt[1:2], gt[0:1] + gt[1:2] + gt[2:3]], axis=0)
    excl4 = s + go - r4
    exr = jnp.concatenate([excl4[g:g + 1] for g in range(4)], axis=1)
    ro = jnp.transpose(exr[:, :_SUB])
    return cs + ro


def _row_counts(blk, tau_ref):
    ge = [jnp.float32(_HW)]
    for b in range(1, _BINS):
        ge.append(jnp.sum(jnp.where(blk >= tau_ref[b], 1.0, 0.0)))
    g = jnp.stack(ge)
    gnext = jnp.concatenate([g[1:], jnp.zeros((1,), jnp.float32)])
    return (g - gnext).astype(jnp.int32).reshape(1, _BINS)


def _scan_body(x_ref, tau_ref, p_ref, c_ref, carry_ref, cnt_ref):
    i = pl.program_id(0)

    @pl.when(i == 0)
    def _init():
        carry_ref[0] = 0.0
        cnt_ref[...] = jnp.zeros((1, _BINS), jnp.int32)

    for r in range(_RBLK):
        blk = x_ref[r]
        p_ref[r] = _row_scan(jnp.tanh(blk)) + carry_ref[0]
        carry_ref[0] = p_ref[r, _SUB - 1, _LANES - 1]
        new_cnt = cnt_ref[...] + _row_counts(blk, tau_ref)
        cnt_ref[...] = new_cnt
        c_ref[r] = new_cnt


def _scan_counts(x_r, tau):
    return pl.pallas_call(
        _scan_body,
        grid=(_ROWS // _RBLK,),
        in_specs=[
            pl.BlockSpec((_RBLK, _SUB, _LANES), lambda i: (i, 0, 0)),
            pl.BlockSpec(memory_space=pltpu.SMEM),
        ],
        out_specs=[
            pl.BlockSpec((_RBLK, _SUB, _LANES), lambda i: (i, 0, 0)),
            pl.BlockSpec((_RBLK, 1, _BINS), lambda i: (i, 0, 0)),
        ],
        out_shape=[
            jax.ShapeDtypeStruct((_ROWS, _SUB, _LANES), jnp.float32),
            jax.ShapeDtypeStruct((_ROWS, 1, _BINS), jnp.int32),
        ],
        scratch_shapes=[
            pltpu.SMEM((1,), jnp.float32),
            pltpu.VMEM((1, _BINS), jnp.int32),
        ],
        compiler_params=pltpu.CompilerParams(
            dimension_semantics=("arbitrary",)),
    )(x_r, tau)


_NW = 24


def _combine_sc(p_flat, idx, msk, cf):
    mesh = plsc.VectorSubcoreMesh(core_axis_name="c", subcore_axis_name="s")

    @functools.partial(
        pl.kernel,
        out_type=jax.ShapeDtypeStruct((_ROWS,), jnp.float32),
        mesh=mesh,
        scratch_types=[
            pltpu.VMEM((16, 128), jnp.int32),
            pltpu.VMEM((16, 128), jnp.float32),
            pltpu.VMEM((8, 128), jnp.float32),
            pltpu.VMEM((16, 128), jnp.float32),
            pltpu.VMEM((16,), jnp.float32),
            pltpu.SemaphoreType.DMA,
        ],
    )
    def kc(p_hbm, idx_hbm, msk_hbm, cf_hbm, out_hbm,
           idx_v, msk_v, cf_v, vals_v, acc_v, sem):
        w = lax.axis_index("s") * 2 + lax.axis_index("c")

        @pl.when(w < _NW)
        def _():
            pltpu.sync_copy(idx_hbm.at[w], idx_v)
            pltpu.sync_copy(msk_hbm.at[w], msk_v)
            pltpu.sync_copy(cf_hbm.at[w], cf_v)
            cps = [pltpu.async_copy(p_hbm.at[idx_v.at[ch]], vals_v.at[ch], sem)
                   for ch in range(16)]
            for cp in cps:
                cp.wait()

            def body(b, acc):
                r = b // 8
                off = (b % 8) * 16
                va = vals_v[r, pl.ds(off, 16)]
                vb = vals_v[r + 8, pl.ds(off, 16)]
                ma = msk_v[r, pl.ds(off, 16)]
                mb = msk_v[r + 8, pl.ds(off, 16)]
                cfv = cf_v[r, pl.ds(off, 16)]
                return acc + cfv * (vb * mb - va * ma)

            acc_v[...] = lax.fori_loop(0, _BINS, body,
                                       jnp.zeros((16,), jnp.float32))
            pltpu.sync_copy(acc_v, out_hbm.at[pl.ds(16 * w, 16)])

    return kc(p_flat, idx, msk, cf)


def _run(x, coeff):
    x_r = x.reshape(_ROWS, _SUB, _LANES)
    mn, mx = _minmax(x_r)
    tau = jnp.linspace(mn[0, 0], mx[0, 0], _BINS + 1)
    P, Ccum = _scan_counts(x_r, tau)
    Q = jnp.concatenate(
        [jnp.zeros((1, _BINS), jnp.int32), Ccum.reshape(_ROWS, _BINS)], axis=0)
    QT = Q.T
    QTw = jnp.stack([lax.slice(QT, (0, 16 * w), (_BINS, 16 * w + 17))
                     for w in range(_NW)])
    qa = QTw[:, :, 0:16].reshape(_NW, 1024)
    qb = QTw[:, :, 1:17].reshape(_NW, 1024)
    qcat = jnp.concatenate([qa, qb], axis=1)
    idx = jnp.maximum(qcat - 1, 0).reshape(_NW, 16, 128)
    msk = (qcat > 0).astype(jnp.float32).reshape(_NW, 16, 128)
    coeffT = coeff.T
    cf = jnp.stack([lax.slice(coeffT, (0, (16 * w) % _C),
                              (_BINS, (16 * w) % _C + 16))
                    for w in range(_NW)]).reshape(_NW, 8, 128)
    z = _combine_sc(P.reshape(-1), idx, msk, cf)
    return z.reshape(_N, _C)


def kernel(x, coeff):
    return _run(x, coeff)
